# baseline (speedup 1.0000x reference)
import functools
import math

import jax
import jax.numpy as jnp
from jax import lax
from jax.experimental import pallas as pl
from jax.experimental.pallas import tpu as pltpu
from jax.experimental.pallas import tpu_sc as plsc

B = 8
KTHETA = 24
KR = 4
IN = 128
ATT = 512
EPS = 1e-6
FOV = math.pi / 2.0
RS = 10.0
NSEG = B * KR * KTHETA
N = 320000

N_PAD = 327680
R_PAD = N_PAD // 128
PT_BLK = 4096
ROW_BLK = PT_BLK // 128
GRID1 = N_PAD // PT_BLK

NSPLIT = 2
PTS_SPLIT = N_PAD // NSPLIT
RS_SPLIT = PTS_SPLIT // 128
BLKS_SPLIT = PTS_SPLIT // PT_BLK

NW = 32
PTS_PER_W = PTS_SPLIT // NW
CHUNK = 1024
SUB = 128
NSUB = CHUNK // SUB
NCHUNK = PTS_PER_W // CHUNK


def _bin_count(v, edges_ref, n_edges):
    cnt = jnp.zeros(v.shape, jnp.int32)
    for k in range(n_edges):
        cnt = cnt + (edges_ref[k, 0] < v).astype(jnp.int32)
    return cnt


def _k1_body(ofs_blk, pc_ref, eth_ref, feat_ref, xyz_ref, b_ref,
             p_ref, flat_ref, g_ref):
    pid = pl.program_id(0) + ofs_blk
    pc = pc_ref[0, 0]
    f = feat_ref[...]
    frow = pid * PT_BLK + lax.broadcasted_iota(jnp.int32, (PT_BLK, 1), 0)
    p = jnp.exp(jnp.log(jnp.abs(f) + EPS) * pc)
    p_ref[...] = jnp.where(frow < N, p, 0.0)

    v = xyz_ref[...]
    ri = lax.broadcasted_iota(jnp.int32, (3 * 128, 128), 0)
    cj = lax.broadcasted_iota(jnp.int32, (3 * 128, 128), 1)
    x = jnp.dot(v, (ri == cj * 3).astype(jnp.float32),
                preferred_element_type=jnp.float32)
    y = jnp.dot(v, (ri == cj * 3 + 1).astype(jnp.float32),
                preferred_element_type=jnp.float32)
    theta = jnp.clip(jnp.arctan2(y, x), -FOV, FOV)
    sec = jnp.clip(_bin_count(theta, eth_ref, KTHETA + 1) - 1, 0, KTHETA - 1)
    r = jnp.maximum(jnp.sqrt(x * x + y * y), EPS)
    rc = ((r > 2.5).astype(jnp.int32) + (r > 5.0).astype(jnp.int32)
          + (r > 7.5).astype(jnp.int32) + (r > 10.0).astype(jnp.int32))
    ring = jnp.minimum(rc, KR - 1)
    grow = pid * ROW_BLK + lax.broadcasted_iota(jnp.int32, (ROW_BLK, 128), 0)
    valid = grow < (N // 128)
    flat = b_ref[...] * (KR * KTHETA) + ring * KTHETA + sec
    flat_ref[...] = jnp.where(valid, flat, 0)
    g_ref[0, :, :] = jnp.where(valid, x, 0.0)
    g_ref[1, :, :] = jnp.where(valid, y, 0.0)
    g_ref[2, :, :] = jnp.where(valid, r, 0.0)
    g_ref[3, :, :] = jnp.where(valid, 1.0, 0.0)


def _stage1(feat, x, batch, pc, eth, split):
    ofs_blk = split * BLKS_SPLIT
    ofs_row = split * (RS_SPLIT // ROW_BLK)
    return pl.pallas_call(
        functools.partial(_k1_body, ofs_blk),
        grid=(BLKS_SPLIT,),
        in_specs=[
            pl.BlockSpec(memory_space=pltpu.SMEM),
            pl.BlockSpec(memory_space=pltpu.SMEM),
            pl.BlockSpec((PT_BLK, 128),
                         lambda i: (jnp.minimum(i + ofs_blk, N // PT_BLK), 0)),
            pl.BlockSpec((ROW_BLK, 3 * 128),
                         lambda i: (jnp.minimum(i + ofs_row, N // 128 // ROW_BLK), 0)),
            pl.BlockSpec((ROW_BLK, 128),
                         lambda i: (jnp.minimum(i + ofs_row, N // 128 // ROW_BLK), 0)),
        ],
        out_specs=[
            pl.BlockSpec((PT_BLK, 128), lambda i: (i, 0)),
            pl.BlockSpec((ROW_BLK, 128), lambda i: (i, 0)),
            pl.BlockSpec((4, ROW_BLK, 128), lambda i: (0, i, 0)),
        ],
        out_shape=[
            jax.ShapeDtypeStruct((PTS_SPLIT, 128), jnp.float32),
            jax.ShapeDtypeStruct((RS_SPLIT, 128), jnp.int32),
            jax.ShapeDtypeStruct((4, RS_SPLIT, 128), jnp.float32),
        ],
    )(pc, eth, feat, x, batch)


NBUF = 4


def _sc_body(p_hbm, flat_hbm, g_hbm, zp_hbm, zg_hbm,
             accp_out, accg_out,
             idx_v, g_v, pbuf,
             accp_s, gx_s, gy_s, gr_s, gn_s,
             sem_gs, *sems):
    sem_pg = sems[:NBUF]
    sem_ps = sems[NBUF:]
    c = lax.axis_index("c")
    s = lax.axis_index("s")
    wid = s * 2 + c

    @pl.when(s == 0)
    def _():
        pltpu.sync_copy(zp_hbm, accp_s)
        pltpu.sync_copy(zg_hbm, gx_s)
        pltpu.sync_copy(zg_hbm, gy_s)
        pltpu.sync_copy(zg_hbm, gr_s)
        pltpu.sync_copy(zg_hbm, gn_s)

    plsc.subcore_barrier()

    base = wid * PTS_PER_W
    frow = wid * (PTS_PER_W // SUB)

    def chunk_body(ci, carry):
        cb = base + ci * CHUNK
        pltpu.sync_copy(flat_hbm.at[pl.ds(frow + ci * NSUB, NSUB)], idx_v)
        pltpu.sync_copy(g_hbm.at[:, pl.ds(frow + ci * NSUB, NSUB)], g_v)
        for jj in range(NSUB):
            idx = idx_v.at[jj]
            pltpu.async_copy(g_v.at[0, jj], gx_s.at[idx], sem_gs, add=True)
            pltpu.async_copy(g_v.at[1, jj], gy_s.at[idx], sem_gs, add=True)
            pltpu.async_copy(g_v.at[2, jj], gr_s.at[idx], sem_gs, add=True)
            pltpu.async_copy(g_v.at[3, jj], gn_s.at[idx], sem_gs, add=True)
        LAG = NBUF - 1
        gd = {}
        sd = {}
        for t in range(NSUB + LAG):
            if t < NSUB:
                b = t % NBUF
                if t >= NBUF:
                    sd[t - NBUF].wait()
                gd[t] = pltpu.async_copy(
                    p_hbm.at[pl.ds(cb + t * SUB, SUB)], pbuf.at[b], sem_pg[b])
            if t >= LAG:
                u = t - LAG
                gd[u].wait()
                sd[u] = pltpu.async_copy(pbuf.at[u % NBUF],
                                         accp_s.at[idx_v.at[u]],
                                         sem_ps[u % NBUF], add=True)
        for u in range(NSUB - NBUF, NSUB):
            sd[u].wait()
        pltpu.make_async_copy(g_hbm.at[:, pl.ds(0, NSUB)], g_v, sem_gs).wait()
        return carry

    lax.fori_loop(0, NCHUNK, chunk_body, 0)

    plsc.subcore_barrier()

    rows = pl.multiple_of(s * 48, 8)
    pltpu.sync_copy(accp_s.at[pl.ds(rows, 48)],
                    accp_out.at[c, pl.ds(rows, 48)])

    @pl.when(s == 0)
    def _():
        pltpu.sync_copy(gx_s, accg_out.at[c, 0])
        pltpu.sync_copy(gy_s, accg_out.at[c, 1])
        pltpu.sync_copy(gr_s, accg_out.at[c, 2])
        pltpu.sync_copy(gn_s, accg_out.at[c, 3])


def _stage2(p, flat2, g2, zp, zg):
    mesh = plsc.VectorSubcoreMesh(core_axis_name="c", subcore_axis_name="s")
    run = functools.partial(
        pl.kernel,
        out_type=[
            jax.ShapeDtypeStruct((2, NSEG, 128), jnp.float32),
            jax.ShapeDtypeStruct((2, 4, NSEG), jnp.float32),
        ],
        mesh=mesh,
        scratch_types=[
            pltpu.VMEM((NSUB, SUB), jnp.int32),
            pltpu.VMEM((4, NSUB, SUB), jnp.float32),
            pltpu.VMEM((NBUF, SUB, 128), jnp.float32),
            pltpu.VMEM_SHARED((NSEG, 128), jnp.float32),
            pltpu.VMEM_SHARED((NSEG,), jnp.float32),
            pltpu.VMEM_SHARED((NSEG,), jnp.float32),
            pltpu.VMEM_SHARED((NSEG,), jnp.float32),
            pltpu.VMEM_SHARED((NSEG,), jnp.float32),
        ] + [pltpu.SemaphoreType.DMA] * (1 + 2 * NBUF),
    )(_sc_body)
    return run(p, flat2, g2, zp, zg)


def _ln(xv, gv, bv):
    m = jnp.mean(xv, axis=-1, keepdims=True)
    v = jnp.mean((xv - m) ** 2, axis=-1, keepdims=True)
    return (xv - m) / jnp.sqrt(v + 1e-5) * gv + bv


def _k3_body(ipc_ref, ap0, ap1, ag0, ag1, w1, b1r, g1, be1, angr, wang, bangr,
             ringr, wring, bringr, wxy, bxyr, gor, bor, out):
    accp = ap0[0] + ap0[1] + ap1[0] + ap1[1]
    g = ag0[0] + ag0[1] + ag1[0] + ag1[1]
    cc = jnp.maximum(g[:, 3:4], 1.0)
    pooled = jnp.maximum(accp / cc, 0.0)
    ipc = ipc_ref[0, 0]
    pooled = jnp.where(pooled > 0.0, jnp.exp(jnp.log(pooled) * ipc), 0.0)

    h = jnp.dot(pooled, w1[...], preferred_element_type=jnp.float32) + b1r[...]
    h = _ln(h, g1[...], be1[...])
    tok = h * 0.5 * (1.0 + lax.erf(h * 0.7071067811865476))

    pe = jnp.dot(angr[...], wang[...], preferred_element_type=jnp.float32)
    pe = pe + bangr[...]
    tok = tok + jnp.broadcast_to(pe[None], (32, KTHETA, ATT)).reshape(NSEG, ATT)

    rpe = jnp.dot(ringr[...], wring[...], preferred_element_type=jnp.float32)
    rpe = rpe + bringr[...]
    tok = tok + jnp.broadcast_to(rpe[None, :, None, :],
                                 (B, KR, KTHETA, ATT)).reshape(NSEG, ATT)

    inv_rs = 1.0 / RS
    mx = g[:, 0:1] / cc
    my = g[:, 1:2] / cc
    mr = g[:, 2:3] / cc
    lc = jnp.log(1.0 + cc) * 0.1
    tok = (tok + (mx * inv_rs) * wxy[0:1, :] + (my * inv_rs) * wxy[1:2, :]
           + (mr * inv_rs) * wxy[2:3, :] + lc * wxy[3:4, :] + bxyr[...])

    out[...] = _ln(tok, gor[...], bor[...])


def _stage3(ipc, ap0, ap1, ag0, ag1, w1, b1, ln1_g, ln1_b, ang, wang, bang,
            ringn, wring, bring, wxy, bxy, lnog, lnob):
    specs = [pl.BlockSpec(memory_space=pltpu.SMEM)]
    specs += [pl.BlockSpec(memory_space=pltpu.VMEM) for _ in range(18)]
    return pl.pallas_call(
        _k3_body,
        in_specs=specs,
        out_specs=pl.BlockSpec(memory_space=pltpu.VMEM),
        out_shape=jax.ShapeDtypeStruct((NSEG, ATT), jnp.float32),
    )(ipc, ap0, ap1, ag0, ag1, w1, b1, ln1_g, ln1_b, ang, wang, bang,
      ringn, wring, bring, wxy, bxy, lnog, lnob)


def kernel(feat, xyz, batch, W1, b1, ln1_g, ln1_b, p, Wang, bang,
           Wring, bring, Wxy, bxy, lnog, lnob):
    featp = feat
    xyzr = xyz.reshape(N // 128, 3 * 128)
    bp = batch.astype(jnp.int32).reshape(N // 128, 128)

    pcv = jnp.maximum(p, 1.0)
    pc = pcv.reshape(1, 1)
    ipc = (1.0 / pcv).reshape(1, 1)
    eth = jnp.linspace(-FOV, FOV, KTHETA + 1,
                       dtype=jnp.float32).reshape(KTHETA + 1, 1)

    zp = jnp.zeros((NSEG, 128), jnp.float32)
    zg = jnp.zeros((NSEG,), jnp.float32)

    accps = []
    accgs = []
    for split in range(NSPLIT):
        pw, flat, g = _stage1(featp, xyzr, bp, pc, eth, split)
        a_p, a_g = _stage2(pw, flat, g, zp, zg)
        accps.append(a_p)
        accgs.append(a_g.transpose(0, 2, 1))

    edges_theta = jnp.linspace(-FOV, FOV, KTHETA + 1, dtype=jnp.float32)
    centers_theta = 0.5 * (edges_theta[:-1] + edges_theta[1:])
    ang = jnp.stack([jnp.sin(centers_theta), jnp.cos(centers_theta)], axis=-1)
    edges_r = jnp.linspace(0.0, RS, KR + 1, dtype=jnp.float32)
    centers_r = 0.5 * (edges_r[:-1] + edges_r[1:])
    ringn = (centers_r / RS).reshape(KR, 1)

    tokens = _stage3(
        ipc, accps[0], accps[1], accgs[0], accgs[1],
        W1, b1.reshape(1, ATT), ln1_g.reshape(1, ATT),
        ln1_b.reshape(1, ATT), ang, Wang, bang.reshape(1, ATT), ringn,
        Wring, bring.reshape(1, ATT), Wxy, bxy.reshape(1, ATT),
        lnog.reshape(1, ATT), lnob.reshape(1, ATT))
    return tokens.reshape(B, KR * KTHETA, ATT)


# speedup vs baseline: 1.5187x; 1.5187x over previous
import functools
import math

import jax
import jax.numpy as jnp
from jax import lax
from jax.experimental import pallas as pl
from jax.experimental.pallas import tpu as pltpu
from jax.experimental.pallas import tpu_sc as plsc

B = 8
KTHETA = 24
KR = 4
IN = 128
ATT = 512
EPS = 1e-6
FOV = math.pi / 2.0
RS = 10.0
NSEG = B * KR * KTHETA
N = 320000

N_PAD = 327680
R_PAD = N_PAD // 128
PT_BLK = 4096
ROW_BLK = PT_BLK // 128
GRID1 = N_PAD // PT_BLK

NSPLIT = 2
PTS_SPLIT = N_PAD // NSPLIT
RS_SPLIT = PTS_SPLIT // 128
BLKS_SPLIT = PTS_SPLIT // PT_BLK

NW = 32
PTS_PER_W = PTS_SPLIT // NW
CHUNK = 1024
SUB = 128
NSUB = CHUNK // SUB
NCHUNK = PTS_PER_W // CHUNK


def _bin_count(v, edges_ref, n_edges):
    cnt = jnp.zeros(v.shape, jnp.int32)
    for k in range(n_edges):
        cnt = cnt + (edges_ref[k, 0] < v).astype(jnp.int32)
    return cnt


def _k1_body(ofs_blk, pc_ref, eth_ref, feat_ref, x_ref, y_ref, b_ref,
             p_ref, flat_ref, g_ref):
    pid = pl.program_id(0) + ofs_blk
    pc = pc_ref[0, 0]
    f = feat_ref[...]
    frow = pid * PT_BLK + lax.broadcasted_iota(jnp.int32, (PT_BLK, 1), 0)
    p = jnp.exp(jnp.log(jnp.abs(f) + EPS) * pc)
    p_ref[...] = jnp.where(frow < N, p, 0.0)

    x = x_ref[...]
    y = y_ref[...]
    theta = jnp.clip(jnp.arctan2(y, x), -FOV, FOV)
    sec = jnp.clip(_bin_count(theta, eth_ref, KTHETA + 1) - 1, 0, KTHETA - 1)
    r = jnp.maximum(jnp.sqrt(x * x + y * y), EPS)
    rc = ((r > 2.5).astype(jnp.int32) + (r > 5.0).astype(jnp.int32)
          + (r > 7.5).astype(jnp.int32) + (r > 10.0).astype(jnp.int32))
    ring = jnp.minimum(rc, KR - 1)
    grow = pid * ROW_BLK + lax.broadcasted_iota(jnp.int32, (ROW_BLK, 128), 0)
    valid = grow < (N // 128)
    flat = b_ref[...] * (KR * KTHETA) + ring * KTHETA + sec
    flat_ref[...] = jnp.where(valid, flat, 0)
    g_ref[0, :, :] = jnp.where(valid, x, 0.0)
    g_ref[1, :, :] = jnp.where(valid, y, 0.0)
    g_ref[2, :, :] = jnp.where(valid, r, 0.0)
    g_ref[3, :, :] = jnp.where(valid, 1.0, 0.0)


def _stage1(feat, x, y, batch, pc, eth, split):
    ofs_blk = split * BLKS_SPLIT
    ofs_row = split * (RS_SPLIT // ROW_BLK)
    return pl.pallas_call(
        functools.partial(_k1_body, ofs_blk),
        grid=(BLKS_SPLIT,),
        in_specs=[
            pl.BlockSpec(memory_space=pltpu.SMEM),
            pl.BlockSpec(memory_space=pltpu.SMEM),
            pl.BlockSpec((PT_BLK, 128),
                         lambda i: (jnp.minimum(i + ofs_blk, N // PT_BLK), 0)),
            pl.BlockSpec((ROW_BLK, 128),
                         lambda i: (jnp.minimum(i + ofs_row, N // 128 // ROW_BLK), 0)),
            pl.BlockSpec((ROW_BLK, 128),
                         lambda i: (jnp.minimum(i + ofs_row, N // 128 // ROW_BLK), 0)),
            pl.BlockSpec((ROW_BLK, 128),
                         lambda i: (jnp.minimum(i + ofs_row, N // 128 // ROW_BLK), 0)),
        ],
        out_specs=[
            pl.BlockSpec((PT_BLK, 128), lambda i: (i, 0)),
            pl.BlockSpec((ROW_BLK, 128), lambda i: (i, 0)),
            pl.BlockSpec((4, ROW_BLK, 128), lambda i: (0, i, 0)),
        ],
        out_shape=[
            jax.ShapeDtypeStruct((PTS_SPLIT, 128), jnp.float32),
            jax.ShapeDtypeStruct((RS_SPLIT, 128), jnp.int32),
            jax.ShapeDtypeStruct((4, RS_SPLIT, 128), jnp.float32),
        ],
    )(pc, eth, feat, x, y, batch)


NBUF = 4


def _sc_body(p_hbm, flat_hbm, g_hbm, zp_hbm, zg_hbm,
             accp_out, accg_out,
             idx_v, g_v, pbuf,
             accp_s, gx_s, gy_s, gr_s, gn_s,
             sem_gs, *sems):
    sem_pg = sems[:NBUF]
    sem_ps = sems[NBUF:]
    c = lax.axis_index("c")
    s = lax.axis_index("s")
    wid = s * 2 + c

    @pl.when(s == 0)
    def _():
        pltpu.sync_copy(zp_hbm, accp_s)
        pltpu.sync_copy(zg_hbm, gx_s)
        pltpu.sync_copy(zg_hbm, gy_s)
        pltpu.sync_copy(zg_hbm, gr_s)
        pltpu.sync_copy(zg_hbm, gn_s)

    plsc.subcore_barrier()

    base = wid * PTS_PER_W
    frow = wid * (PTS_PER_W // SUB)

    def chunk_body(ci, carry):
        cb = base + ci * CHUNK
        pltpu.sync_copy(flat_hbm.at[pl.ds(frow + ci * NSUB, NSUB)], idx_v)
        pltpu.sync_copy(g_hbm.at[:, pl.ds(frow + ci * NSUB, NSUB)], g_v)
        for jj in range(NSUB):
            idx = idx_v.at[jj]
            pltpu.async_copy(g_v.at[0, jj], gx_s.at[idx], sem_gs, add=True)
            pltpu.async_copy(g_v.at[1, jj], gy_s.at[idx], sem_gs, add=True)
            pltpu.async_copy(g_v.at[2, jj], gr_s.at[idx], sem_gs, add=True)
            pltpu.async_copy(g_v.at[3, jj], gn_s.at[idx], sem_gs, add=True)
        LAG = NBUF - 1
        gd = {}
        sd = {}
        for t in range(NSUB + LAG):
            if t < NSUB:
                b = t % NBUF
                if t >= NBUF:
                    sd[t - NBUF].wait()
                gd[t] = pltpu.async_copy(
                    p_hbm.at[pl.ds(cb + t * SUB, SUB)], pbuf.at[b], sem_pg[b])
            if t >= LAG:
                u = t - LAG
                gd[u].wait()
                sd[u] = pltpu.async_copy(pbuf.at[u % NBUF],
                                         accp_s.at[idx_v.at[u]],
                                         sem_ps[u % NBUF], add=True)
        for u in range(NSUB - NBUF, NSUB):
            sd[u].wait()
        pltpu.make_async_copy(g_hbm.at[:, pl.ds(0, NSUB)], g_v, sem_gs).wait()
        return carry

    lax.fori_loop(0, NCHUNK, chunk_body, 0)

    plsc.subcore_barrier()

    rows = pl.multiple_of(s * 48, 8)
    pltpu.sync_copy(accp_s.at[pl.ds(rows, 48)],
                    accp_out.at[c, pl.ds(rows, 48)])

    @pl.when(s == 0)
    def _():
        pltpu.sync_copy(gx_s, accg_out.at[c, 0])
        pltpu.sync_copy(gy_s, accg_out.at[c, 1])
        pltpu.sync_copy(gr_s, accg_out.at[c, 2])
        pltpu.sync_copy(gn_s, accg_out.at[c, 3])


def _stage2(p, flat2, g2, zp, zg):
    mesh = plsc.VectorSubcoreMesh(core_axis_name="c", subcore_axis_name="s")
    run = functools.partial(
        pl.kernel,
        out_type=[
            jax.ShapeDtypeStruct((2, NSEG, 128), jnp.float32),
            jax.ShapeDtypeStruct((2, 4, NSEG), jnp.float32),
        ],
        mesh=mesh,
        scratch_types=[
            pltpu.VMEM((NSUB, SUB), jnp.int32),
            pltpu.VMEM((4, NSUB, SUB), jnp.float32),
            pltpu.VMEM((NBUF, SUB, 128), jnp.float32),
            pltpu.VMEM_SHARED((NSEG, 128), jnp.float32),
            pltpu.VMEM_SHARED((NSEG,), jnp.float32),
            pltpu.VMEM_SHARED((NSEG,), jnp.float32),
            pltpu.VMEM_SHARED((NSEG,), jnp.float32),
            pltpu.VMEM_SHARED((NSEG,), jnp.float32),
        ] + [pltpu.SemaphoreType.DMA] * (1 + 2 * NBUF),
    )(_sc_body)
    return run(p, flat2, g2, zp, zg)


def _ln(xv, gv, bv):
    m = jnp.mean(xv, axis=-1, keepdims=True)
    v = jnp.mean((xv - m) ** 2, axis=-1, keepdims=True)
    return (xv - m) / jnp.sqrt(v + 1e-5) * gv + bv


def _k3_body(ipc_ref, ap0, ap1, ag0, ag1, w1, b1r, g1, be1, angr, wang, bangr,
             ringr, wring, bringr, wxy, bxyr, gor, bor, out):
    accp = ap0[0] + ap0[1] + ap1[0] + ap1[1]
    g = ag0[0] + ag0[1] + ag1[0] + ag1[1]
    cc = jnp.maximum(g[:, 3:4], 1.0)
    pooled = jnp.maximum(accp / cc, 0.0)
    ipc = ipc_ref[0, 0]
    pooled = jnp.where(pooled > 0.0, jnp.exp(jnp.log(pooled) * ipc), 0.0)

    h = jnp.dot(pooled, w1[...], preferred_element_type=jnp.float32) + b1r[...]
    h = _ln(h, g1[...], be1[...])
    tok = h * 0.5 * (1.0 + lax.erf(h * 0.7071067811865476))

    pe = jnp.dot(angr[...], wang[...], preferred_element_type=jnp.float32)
    pe = pe + bangr[...]
    tok = tok + jnp.broadcast_to(pe[None], (32, KTHETA, ATT)).reshape(NSEG, ATT)

    rpe = jnp.dot(ringr[...], wring[...], preferred_element_type=jnp.float32)
    rpe = rpe + bringr[...]
    tok = tok + jnp.broadcast_to(rpe[None, :, None, :],
                                 (B, KR, KTHETA, ATT)).reshape(NSEG, ATT)

    inv_rs = 1.0 / RS
    mx = g[:, 0:1] / cc
    my = g[:, 1:2] / cc
    mr = g[:, 2:3] / cc
    lc = jnp.log(1.0 + cc) * 0.1
    tok = (tok + (mx * inv_rs) * wxy[0:1, :] + (my * inv_rs) * wxy[1:2, :]
           + (mr * inv_rs) * wxy[2:3, :] + lc * wxy[3:4, :] + bxyr[...])

    out[...] = _ln(tok, gor[...], bor[...])


def _stage3(ipc, ap0, ap1, ag0, ag1, w1, b1, ln1_g, ln1_b, ang, wang, bang,
            ringn, wring, bring, wxy, bxy, lnog, lnob):
    specs = [pl.BlockSpec(memory_space=pltpu.SMEM)]
    specs += [pl.BlockSpec(memory_space=pltpu.VMEM) for _ in range(18)]
    return pl.pallas_call(
        _k3_body,
        in_specs=specs,
        out_specs=pl.BlockSpec(memory_space=pltpu.VMEM),
        out_shape=jax.ShapeDtypeStruct((NSEG, ATT), jnp.float32),
    )(ipc, ap0, ap1, ag0, ag1, w1, b1, ln1_g, ln1_b, ang, wang, bang,
      ringn, wring, bring, wxy, bxy, lnog, lnob)


def kernel(feat, xyz, batch, W1, b1, ln1_g, ln1_b, p, Wang, bang,
           Wring, bring, Wxy, bxy, lnog, lnob):
    featp = feat
    xp = xyz[:, 0].reshape(N // 128, 128)
    yp = xyz[:, 1].reshape(N // 128, 128)
    bp = batch.astype(jnp.int32).reshape(N // 128, 128)

    pcv = jnp.maximum(p, 1.0)
    pc = pcv.reshape(1, 1)
    ipc = (1.0 / pcv).reshape(1, 1)
    eth = jnp.linspace(-FOV, FOV, KTHETA + 1,
                       dtype=jnp.float32).reshape(KTHETA + 1, 1)

    zp = jnp.zeros((NSEG, 128), jnp.float32)
    zg = jnp.zeros((NSEG,), jnp.float32)

    accps = []
    accgs = []
    for split in range(NSPLIT):
        pw, flat, g = _stage1(featp, xp, yp, bp, pc, eth, split)
        a_p, a_g = _stage2(pw, flat, g, zp, zg)
        accps.append(a_p)
        accgs.append(a_g.transpose(0, 2, 1))

    edges_theta = jnp.linspace(-FOV, FOV, KTHETA + 1, dtype=jnp.float32)
    centers_theta = 0.5 * (edges_theta[:-1] + edges_theta[1:])
    ang = jnp.stack([jnp.sin(centers_theta), jnp.cos(centers_theta)], axis=-1)
    edges_r = jnp.linspace(0.0, RS, KR + 1, dtype=jnp.float32)
    centers_r = 0.5 * (edges_r[:-1] + edges_r[1:])
    ringn = (centers_r / RS).reshape(KR, 1)

    tokens = _stage3(
        ipc, accps[0], accps[1], accgs[0], accgs[1],
        W1, b1.reshape(1, ATT), ln1_g.reshape(1, ATT),
        ln1_b.reshape(1, ATT), ang, Wang, bang.reshape(1, ATT), ringn,
        Wring, bring.reshape(1, ATT), Wxy, bxy.reshape(1, ATT),
        lnog.reshape(1, ATT), lnob.reshape(1, ATT))
    return tokens.reshape(B, KR * KTHETA, ATT)
